# Initial kernel scaffold; baseline (speedup 1.0000x reference)
import jax
import jax.numpy as jnp
from jax.experimental import pallas as pl


def kernel(x, edge_index, batch, W1, b1, g1, be1, W2, b2, g2, be2, W3, b3, g3, be3, Wc, bc):
    raise NotImplementedError("write your pallas kernel here")


# speedup vs baseline: 11.3862x; 11.3862x over previous
import functools

import jax
import jax.numpy as jnp
from jax import lax
from jax.experimental import pallas as pl
from jax.experimental.pallas import tpu as pltpu
from jax.experimental.pallas import tpu_sc as plsc

_NC = 2
_NS = 16
_LW = 16
_CH = 128


def _sc_scatter(hs, src, dst):
  n, h = hs.shape
  e = src.shape[0]
  nw = _NC * _NS
  epw = e // nw
  nfull = epw // _CH
  tail = epw - nfull * _CH
  rpt8 = (-(-n // _NS) + 7) // 8 * 8
  last = n - (_NS - 1) * rpt8
  mesh = plsc.VectorSubcoreMesh(core_axis_name="c", subcore_axis_name="s")

  scratch = [
      pltpu.VMEM((_CH,), jnp.int32),
      pltpu.VMEM((_CH,), jnp.int32),
      pltpu.VMEM((_CH, h), jnp.float32),
      pltpu.VMEM_SHARED((n, h), jnp.float32),
      pltpu.SemaphoreType.DMA,
  ]
  if tail:
    scratch += [pltpu.VMEM((tail,), jnp.int32),
                pltpu.VMEM((tail,), jnp.int32),
                pltpu.VMEM((tail, h), jnp.float32)]

  @functools.partial(
      pl.kernel,
      out_type=jax.ShapeDtypeStruct((_NC * n, h), jnp.float32),
      mesh=mesh,
      scratch_types=scratch,
  )
  def k(hs_hbm, src_hbm, dst_hbm, out_hbm, isrc, idst, rows, acc, sem, *tl):
    c = lax.axis_index("c")
    s = lax.axis_index("s")
    base = (c * _NS + s) * epw

    def stage(src_ref, dst_ref, src0, dst0, nrow):
      for r in range(0, nrow, _CH):
        m = min(_CH, nrow - r)
        pltpu.sync_copy(src_ref.at[pl.ds(src0 + r, m)], rows.at[pl.ds(0, m)])
        pltpu.sync_copy(rows.at[pl.ds(0, m)], dst_ref.at[pl.ds(dst0 + r, m)])

    @pl.when(s < _NS - 1)
    def _():
      stage(hs_hbm, acc, s * rpt8, s * rpt8, rpt8)

    @pl.when(s == _NS - 1)
    def _():
      stage(hs_hbm, acc, (_NS - 1) * rpt8, (_NS - 1) * rpt8, last)

    plsc.subcore_barrier()

    def chunk(k_, carry):
      off = base + k_ * _CH
      pltpu.sync_copy(src_hbm.at[pl.ds(off, _CH)], isrc)
      pltpu.sync_copy(dst_hbm.at[pl.ds(off, _CH)], idst)
      pltpu.async_copy(hs_hbm.at[isrc], rows, sem).wait()
      pltpu.sync_copy(rows, acc.at[idst], add=True)
      return carry
    lax.fori_loop(0, nfull, chunk, 0)

    if tail:
      isrc_t, idst_t, rows_t = tl
      off = base + nfull * _CH
      pltpu.sync_copy(src_hbm.at[pl.ds(off, tail)], isrc_t)
      pltpu.sync_copy(dst_hbm.at[pl.ds(off, tail)], idst_t)
      pltpu.async_copy(hs_hbm.at[isrc_t], rows_t, sem).wait()
      pltpu.sync_copy(rows_t, acc.at[idst_t], add=True)

    plsc.subcore_barrier()

    @pl.when(s < _NS - 1)
    def _():
      stage(acc, out_hbm, s * rpt8, c * n + s * rpt8, rpt8)

    @pl.when(s == _NS - 1)
    def _():
      stage(acc, out_hbm, (_NS - 1) * rpt8, c * n + (_NS - 1) * rpt8, last)

  return k(hs, src, dst)


_BN = 2000


def _tc_prep(deg2, x, w1):
  n, d = x.shape
  h = w1.shape[1]
  nb = n // _BN

  def body(deg_ref, x_ref, w_ref, hs_ref, dis_ref):
    deg = deg_ref[0] + deg_ref[1] - 1.0
    dis = lax.rsqrt(deg)
    dis_ref[...] = dis[:, :_LW]
    hm = jnp.dot(x_ref[...], w_ref[...], preferred_element_type=jnp.float32)
    hs_ref[...] = hm * dis

  return pl.pallas_call(
      body,
      grid=(nb,),
      in_specs=[
          pl.BlockSpec((2, _BN, h), lambda i: (0, i, 0)),
          pl.BlockSpec((_BN, d), lambda i: (i, 0)),
          pl.BlockSpec((d, h), lambda i: (0, 0)),
      ],
      out_specs=[
          pl.BlockSpec((_BN, h), lambda i: (i, 0)),
          pl.BlockSpec((_BN, _LW), lambda i: (i, 0)),
      ],
      out_shape=[
          jax.ShapeDtypeStruct((n, h), jnp.float32),
          jax.ShapeDtypeStruct((n, _LW), jnp.float32),
      ],
  )(deg2, x, w1)


def _tc_layer(s2, hs, dis, b, g, be, wn):
  n, h = hs.shape
  nb = n // _BN

  def body(s2_ref, hs_ref, dis_ref, b_ref, g_ref, be_ref, wn_ref,
           out_ref, ystore, stats):
    i = pl.program_id(0)

    @pl.when(i == 0)
    def _():
      stats[...] = jnp.zeros_like(stats)
      out_ref[...] = jnp.zeros_like(out_ref)

    @pl.when(i < nb)
    def _():
      y = (dis_ref[...][:, 0:1] * (s2_ref[0] + s2_ref[1] - hs_ref[...])
           + b_ref[...])
      ystore[pl.ds(i * _BN, _BN), :] = y
      stats[0:1, :] += jnp.sum(y, axis=0, keepdims=True)
      stats[1:2, :] += jnp.sum(y * y, axis=0, keepdims=True)

    @pl.when(i >= nb)
    def _():
      @pl.when(i == nb)
      def _():
        mean = stats[0:1, :] / n
        var = stats[1:2, :] / n - mean * mean
        stats[2:3, :] = mean
        stats[3:4, :] = lax.rsqrt(var + 1e-5)

      j = i - nb
      y = ystore[pl.ds(j * _BN, _BN), :]
      xp = jnp.maximum((y - stats[2:3, :]) * stats[3:4, :] * g_ref[...]
                       + be_ref[...], 0.0)
      out_ref[...] = dis_ref[...][:, 0:1] * jnp.dot(
          xp, wn_ref[...], preferred_element_type=jnp.float32)

  return pl.pallas_call(
      body,
      grid=(2 * nb,),
      in_specs=[
          pl.BlockSpec((2, _BN, h), lambda i: (0, jnp.minimum(i, nb - 1), 0)),
          pl.BlockSpec((_BN, h), lambda i: (jnp.minimum(i, nb - 1), 0)),
          pl.BlockSpec((_BN, _LW),
                       lambda i: (jnp.where(i < nb, i, i - nb), 0)),
          pl.BlockSpec((1, h), lambda i: (0, 0)),
          pl.BlockSpec((1, h), lambda i: (0, 0)),
          pl.BlockSpec((1, h), lambda i: (0, 0)),
          pl.BlockSpec((h, h), lambda i: (0, 0)),
      ],
      out_specs=pl.BlockSpec((_BN, h), lambda i: (jnp.maximum(i - nb, 0), 0)),
      out_shape=jax.ShapeDtypeStruct((n, h), jnp.float32),
      scratch_shapes=[
          pltpu.VMEM((n, h), jnp.float32),
          pltpu.VMEM((8, h), jnp.float32),
      ],
  )(s2, hs, dis, b, g, be, wn)


def _tc_final(s2, hs, dis, b, g, be, batch2, wc, bc):
  n, h = hs.shape
  nb = n // _BN
  ng = 16
  co = wc.shape[1]

  def body(s2_ref, hs_ref, dis_ref, b_ref, g_ref, be_ref, batch_ref,
           wc_ref, bc_ref, out_ref, ystore, stats, psum, pcnt):
    i = pl.program_id(0)

    @pl.when(i == 0)
    def _():
      stats[...] = jnp.zeros_like(stats)
      psum[...] = jnp.zeros_like(psum)
      pcnt[...] = jnp.zeros_like(pcnt)
      out_ref[...] = jnp.zeros_like(out_ref)

    @pl.when(i < nb)
    def _():
      y = (dis_ref[...][:, 0:1] * (s2_ref[0] + s2_ref[1] - hs_ref[...])
           + b_ref[...])
      ystore[pl.ds(i * _BN, _BN), :] = y
      stats[0:1, :] += jnp.sum(y, axis=0, keepdims=True)
      stats[1:2, :] += jnp.sum(y * y, axis=0, keepdims=True)

    @pl.when(i >= nb)
    def _():
      @pl.when(i == nb)
      def _():
        mean = stats[0:1, :] / n
        var = stats[1:2, :] / n - mean * mean
        stats[2:3, :] = mean
        stats[3:4, :] = lax.rsqrt(var + 1e-5)

      j = i - nb
      y = ystore[pl.ds(j * _BN, _BN), :]
      xp = jnp.maximum((y - stats[2:3, :]) * stats[3:4, :] * g_ref[...]
                       + be_ref[...], 0.0)
      oh = (batch_ref[...] == lax.broadcasted_iota(jnp.int32, (1, ng), 1)
            ).astype(jnp.float32)
      psum[...] += lax.dot_general(oh, xp, (((0,), (0,)), ((), ())),
                                   preferred_element_type=jnp.float32)
      ones = jnp.ones((_BN, 1), jnp.float32)
      pcnt[...] += lax.dot_general(oh, ones, (((0,), (0,)), ((), ())),
                                   preferred_element_type=jnp.float32)

      @pl.when(i == 2 * nb - 1)
      def _():
        p = psum[...] / jnp.maximum(pcnt[...], 1.0)
        out_ref[...] = jnp.dot(p, wc_ref[...],
                               preferred_element_type=jnp.float32) + bc_ref[...]

  return pl.pallas_call(
      body,
      grid=(2 * nb,),
      in_specs=[
          pl.BlockSpec((2, _BN, h), lambda i: (0, jnp.minimum(i, nb - 1), 0)),
          pl.BlockSpec((_BN, h), lambda i: (jnp.minimum(i, nb - 1), 0)),
          pl.BlockSpec((_BN, _LW),
                       lambda i: (jnp.where(i < nb, i, i - nb), 0)),
          pl.BlockSpec((1, h), lambda i: (0, 0)),
          pl.BlockSpec((1, h), lambda i: (0, 0)),
          pl.BlockSpec((1, h), lambda i: (0, 0)),
          pl.BlockSpec((_BN, 1), lambda i: (jnp.maximum(i - nb, 0), 0)),
          pl.BlockSpec((h, co), lambda i: (0, 0)),
          pl.BlockSpec((1, co), lambda i: (0, 0)),
      ],
      out_specs=pl.BlockSpec((ng, co), lambda i: (0, 0)),
      out_shape=jax.ShapeDtypeStruct((ng, co), jnp.float32),
      scratch_shapes=[
          pltpu.VMEM((n, h), jnp.float32),
          pltpu.VMEM((8, h), jnp.float32),
          pltpu.VMEM((ng, h), jnp.float32),
          pltpu.VMEM((ng, h), jnp.float32),
      ],
  )(s2, hs, dis, b, g, be, batch2, wc, bc)


def kernel(x, edge_index, batch, W1, b1, g1, be1, W2, b2, g2, be2,
           W3, b3, g3, be3, Wc, bc):
  n, d = x.shape
  h = W1.shape[1]
  src_i = edge_index[0]
  dst_i = edge_index[1]

  ones_tab = jnp.ones((n, h), jnp.float32)
  degp = _sc_scatter(ones_tab, src_i, dst_i).reshape(_NC, n, h)
  hs1, dis = _tc_prep(degp, x, W1)

  s1 = _sc_scatter(hs1, src_i, dst_i).reshape(_NC, n, h)
  hs2 = _tc_layer(s1, hs1, dis, b1.reshape(1, h), g1.reshape(1, h),
                  be1.reshape(1, h), W2)

  s2m = _sc_scatter(hs2, src_i, dst_i).reshape(_NC, n, h)
  hs3 = _tc_layer(s2m, hs2, dis, b2.reshape(1, h), g2.reshape(1, h),
                  be2.reshape(1, h), W3)

  s3m = _sc_scatter(hs3, src_i, dst_i).reshape(_NC, n, h)
  logits = _tc_final(s3m, hs3, dis, b3.reshape(1, h), g3.reshape(1, h),
                     be3.reshape(1, h), batch.reshape(n, 1), Wc,
                     bc.reshape(1, Wc.shape[1]))
  return logits


# speedup vs baseline: 19.6628x; 1.7269x over previous
import functools

import jax
import jax.numpy as jnp
from jax import lax
from jax.experimental import pallas as pl
from jax.experimental.pallas import tpu as pltpu
from jax.experimental.pallas import tpu_sc as plsc

_NC = 2
_NS = 16
_LW = 16
_CH = 128


def _sc_scatter(hs, src, dst):
  n, h = hs.shape
  e = src.shape[0]
  nw = _NC * _NS
  epw = e // nw
  nfull = epw // _CH
  tail = epw - nfull * _CH
  rpt8 = (-(-n // _NS) + 7) // 8 * 8
  last = n - (_NS - 1) * rpt8
  mesh = plsc.VectorSubcoreMesh(core_axis_name="c", subcore_axis_name="s")

  scratch = [
      pltpu.VMEM((_CH,), jnp.int32),
      pltpu.VMEM((_CH,), jnp.int32),
      pltpu.VMEM((_CH,), jnp.int32),
      pltpu.VMEM((_CH,), jnp.int32),
      pltpu.VMEM((_CH, h), jnp.float32),
      pltpu.VMEM((_CH, h), jnp.float32),
      pltpu.VMEM_SHARED((n, h), jnp.float32),
      pltpu.SemaphoreType.DMA,
      pltpu.SemaphoreType.DMA,
  ]
  if tail:
    scratch += [pltpu.VMEM((tail,), jnp.int32),
                pltpu.VMEM((tail,), jnp.int32),
                pltpu.VMEM((tail, h), jnp.float32)]

  npair = nfull // 2
  extra = nfull - 2 * npair

  @functools.partial(
      pl.kernel,
      out_type=jax.ShapeDtypeStruct((_NC * n, h), jnp.float32),
      mesh=mesh,
      scratch_types=scratch,
  )
  def k(hs_hbm, src_hbm, dst_hbm, out_hbm, ia_s, ia_d, ib_s, ib_d,
        rows_a, rows_b, acc, sem_a, sem_b, *tl):
    c = lax.axis_index("c")
    s = lax.axis_index("s")
    base = (c * _NS + s) * epw

    def stage(src_ref, dst_ref, src0, dst0, nrow):
      for r in range(0, nrow, _CH):
        m = min(_CH, nrow - r)
        pltpu.sync_copy(src_ref.at[pl.ds(src0 + r, m)], rows_a.at[pl.ds(0, m)])
        pltpu.sync_copy(rows_a.at[pl.ds(0, m)], dst_ref.at[pl.ds(dst0 + r, m)])

    def load_idx(off, bs, bd):
      pltpu.sync_copy(src_hbm.at[pl.ds(off, _CH)], bs)
      pltpu.sync_copy(dst_hbm.at[pl.ds(off, _CH)], bd)

    @pl.when(s < _NS - 1)
    def _():
      stage(hs_hbm, acc, s * rpt8, s * rpt8, rpt8)

    @pl.when(s == _NS - 1)
    def _():
      stage(hs_hbm, acc, (_NS - 1) * rpt8, (_NS - 1) * rpt8, last)

    plsc.subcore_barrier()

    if npair:
      load_idx(base, ia_s, ia_d)
      pltpu.async_copy(hs_hbm.at[ia_s], rows_a, sem_a)

      def pair(kk, carry):
        k0 = 2 * kk
        load_idx(base + (k0 + 1) * _CH, ib_s, ib_d)
        pltpu.async_copy(hs_hbm.at[ib_s], rows_b, sem_b)
        pltpu.make_async_copy(hs_hbm.at[ia_s], rows_a, sem_a).wait()
        pltpu.sync_copy(rows_a, acc.at[ia_d], add=True)

        @pl.when(kk < npair - 1)
        def _():
          load_idx(base + (k0 + 2) * _CH, ia_s, ia_d)
          pltpu.async_copy(hs_hbm.at[ia_s], rows_a, sem_a)

        pltpu.make_async_copy(hs_hbm.at[ib_s], rows_b, sem_b).wait()
        pltpu.sync_copy(rows_b, acc.at[ib_d], add=True)
        return carry
      lax.fori_loop(0, npair, pair, 0)

    if extra:
      off = base + 2 * npair * _CH
      load_idx(off, ia_s, ia_d)
      pltpu.async_copy(hs_hbm.at[ia_s], rows_a, sem_a).wait()
      pltpu.sync_copy(rows_a, acc.at[ia_d], add=True)

    if tail:
      isrc_t, idst_t, rows_t = tl
      off = base + nfull * _CH
      pltpu.sync_copy(src_hbm.at[pl.ds(off, tail)], isrc_t)
      pltpu.sync_copy(dst_hbm.at[pl.ds(off, tail)], idst_t)
      pltpu.async_copy(hs_hbm.at[isrc_t], rows_t, sem_a).wait()
      pltpu.sync_copy(rows_t, acc.at[idst_t], add=True)

    plsc.subcore_barrier()

    @pl.when(s < _NS - 1)
    def _():
      stage(acc, out_hbm, s * rpt8, c * n + s * rpt8, rpt8)

    @pl.when(s == _NS - 1)
    def _():
      stage(acc, out_hbm, (_NS - 1) * rpt8, c * n + (_NS - 1) * rpt8, last)

  return k(hs, src, dst)


def _sc_count(ones_tab, dst):
  n, h = ones_tab.shape
  e = dst.shape[0]
  nw = _NC * _NS
  epw = e // nw
  nfull = epw // _CH
  tail = epw - nfull * _CH
  rpt8 = (-(-n // _NS) + 7) // 8 * 8
  last = n - (_NS - 1) * rpt8
  npair = nfull // 2
  extra = nfull - 2 * npair
  mesh = plsc.VectorSubcoreMesh(core_axis_name="c", subcore_axis_name="s")

  scratch = [
      pltpu.VMEM((_CH,), jnp.int32),
      pltpu.VMEM((_CH,), jnp.int32),
      pltpu.VMEM((_CH, h), jnp.float32),
      pltpu.VMEM_SHARED((n, h), jnp.float32),
      pltpu.SemaphoreType.DMA,
      pltpu.SemaphoreType.DMA,
  ]
  if tail:
    scratch += [pltpu.VMEM((tail,), jnp.int32)]

  @functools.partial(
      pl.kernel,
      out_type=jax.ShapeDtypeStruct((_NC * n, h), jnp.float32),
      mesh=mesh,
      scratch_types=scratch,
  )
  def k(ones_hbm, dst_hbm, out_hbm, ia, ib, ones_v, acc, sem_a, sem_b, *tl):
    c = lax.axis_index("c")
    s = lax.axis_index("s")
    base = (c * _NS + s) * epw

    def stage(src_ref, dst_ref, src0, dst0, nrow):
      for r in range(0, nrow, _CH):
        m = min(_CH, nrow - r)
        pltpu.sync_copy(src_ref.at[pl.ds(src0 + r, m)], ones_v.at[pl.ds(0, m)])
        pltpu.sync_copy(ones_v.at[pl.ds(0, m)], dst_ref.at[pl.ds(dst0 + r, m)])

    @pl.when(s < _NS - 1)
    def _():
      stage(ones_hbm, acc, s * rpt8, s * rpt8, rpt8)

    @pl.when(s == _NS - 1)
    def _():
      stage(ones_hbm, acc, (_NS - 1) * rpt8, (_NS - 1) * rpt8, last)

    pltpu.sync_copy(ones_hbm.at[pl.ds(0, _CH)], ones_v)
    plsc.subcore_barrier()

    if npair:
      pltpu.async_copy(dst_hbm.at[pl.ds(base, _CH)], ia, sem_a)

      def pair(kk, carry):
        k0 = 2 * kk
        pltpu.async_copy(dst_hbm.at[pl.ds(base + (k0 + 1) * _CH, _CH)], ib,
                         sem_b)
        pltpu.make_async_copy(dst_hbm.at[pl.ds(base, _CH)], ia, sem_a).wait()
        pltpu.sync_copy(ones_v, acc.at[ia], add=True)

        @pl.when(kk < npair - 1)
        def _():
          pltpu.async_copy(dst_hbm.at[pl.ds(base + (k0 + 2) * _CH, _CH)], ia,
                           sem_a)

        pltpu.make_async_copy(dst_hbm.at[pl.ds(base, _CH)], ib, sem_b).wait()
        pltpu.sync_copy(ones_v, acc.at[ib], add=True)
        return carry
      lax.fori_loop(0, npair, pair, 0)

    if extra:
      pltpu.sync_copy(dst_hbm.at[pl.ds(base + 2 * npair * _CH, _CH)], ia)
      pltpu.sync_copy(ones_v, acc.at[ia], add=True)

    if tail:
      idx_t, = tl
      pltpu.sync_copy(dst_hbm.at[pl.ds(base + nfull * _CH, tail)], idx_t)
      pltpu.sync_copy(ones_v.at[pl.ds(0, tail)], acc.at[idx_t], add=True)

    plsc.subcore_barrier()

    @pl.when(s < _NS - 1)
    def _():
      stage(acc, out_hbm, s * rpt8, c * n + s * rpt8, rpt8)

    @pl.when(s == _NS - 1)
    def _():
      stage(acc, out_hbm, (_NS - 1) * rpt8, c * n + (_NS - 1) * rpt8, last)

  return k(ones_tab, dst)


_BN = 2000


def _tc_prep(deg2, x, w1):
  n, d = x.shape
  h = w1.shape[1]
  nb = n // _BN

  def body(deg_ref, x_ref, w_ref, hs_ref, dis_ref):
    deg = deg_ref[0] + deg_ref[1] - 1.0
    dis = lax.rsqrt(deg)
    dis_ref[...] = dis[:, :_LW]
    hm = jnp.dot(x_ref[...], w_ref[...], preferred_element_type=jnp.float32)
    hs_ref[...] = hm * dis

  return pl.pallas_call(
      body,
      grid=(nb,),
      in_specs=[
          pl.BlockSpec((2, _BN, h), lambda i: (0, i, 0)),
          pl.BlockSpec((_BN, d), lambda i: (i, 0)),
          pl.BlockSpec((d, h), lambda i: (0, 0)),
      ],
      out_specs=[
          pl.BlockSpec((_BN, h), lambda i: (i, 0)),
          pl.BlockSpec((_BN, _LW), lambda i: (i, 0)),
      ],
      out_shape=[
          jax.ShapeDtypeStruct((n, h), jnp.float32),
          jax.ShapeDtypeStruct((n, _LW), jnp.float32),
      ],
  )(deg2, x, w1)


def _tc_layer(s2, hs, dis, b, g, be, wn):
  n, h = hs.shape
  nb = n // _BN

  def body(s2_ref, hs_ref, dis_ref, b_ref, g_ref, be_ref, wn_ref,
           out_ref, ystore, stats):
    i = pl.program_id(0)

    @pl.when(i == 0)
    def _():
      stats[...] = jnp.zeros_like(stats)
      out_ref[...] = jnp.zeros_like(out_ref)

    @pl.when(i < nb)
    def _():
      y = (dis_ref[...][:, 0:1] * (s2_ref[0] + s2_ref[1] - hs_ref[...])
           + b_ref[...])
      ystore[pl.ds(i * _BN, _BN), :] = y
      stats[0:1, :] += jnp.sum(y, axis=0, keepdims=True)
      stats[1:2, :] += jnp.sum(y * y, axis=0, keepdims=True)

    @pl.when(i >= nb)
    def _():
      @pl.when(i == nb)
      def _():
        mean = stats[0:1, :] / n
        var = stats[1:2, :] / n - mean * mean
        stats[2:3, :] = mean
        stats[3:4, :] = lax.rsqrt(var + 1e-5)

      j = i - nb
      y = ystore[pl.ds(j * _BN, _BN), :]
      xp = jnp.maximum((y - stats[2:3, :]) * stats[3:4, :] * g_ref[...]
                       + be_ref[...], 0.0)
      out_ref[...] = dis_ref[...][:, 0:1] * jnp.dot(
          xp, wn_ref[...], preferred_element_type=jnp.float32)

  return pl.pallas_call(
      body,
      grid=(2 * nb,),
      in_specs=[
          pl.BlockSpec((2, _BN, h), lambda i: (0, jnp.minimum(i, nb - 1), 0)),
          pl.BlockSpec((_BN, h), lambda i: (jnp.minimum(i, nb - 1), 0)),
          pl.BlockSpec((_BN, _LW),
                       lambda i: (jnp.where(i < nb, i, i - nb), 0)),
          pl.BlockSpec((1, h), lambda i: (0, 0)),
          pl.BlockSpec((1, h), lambda i: (0, 0)),
          pl.BlockSpec((1, h), lambda i: (0, 0)),
          pl.BlockSpec((h, h), lambda i: (0, 0)),
      ],
      out_specs=pl.BlockSpec((_BN, h), lambda i: (jnp.maximum(i - nb, 0), 0)),
      out_shape=jax.ShapeDtypeStruct((n, h), jnp.float32),
      scratch_shapes=[
          pltpu.VMEM((n, h), jnp.float32),
          pltpu.VMEM((8, h), jnp.float32),
      ],
  )(s2, hs, dis, b, g, be, wn)


def _tc_final(s2, hs, dis, b, g, be, batch2, wc, bc):
  n, h = hs.shape
  nb = n // _BN
  ng = 16
  co = wc.shape[1]

  def body(s2_ref, hs_ref, dis_ref, b_ref, g_ref, be_ref, batch_ref,
           wc_ref, bc_ref, out_ref, ystore, stats, psum, pcnt):
    i = pl.program_id(0)

    @pl.when(i == 0)
    def _():
      stats[...] = jnp.zeros_like(stats)
      psum[...] = jnp.zeros_like(psum)
      pcnt[...] = jnp.zeros_like(pcnt)
      out_ref[...] = jnp.zeros_like(out_ref)

    @pl.when(i < nb)
    def _():
      y = (dis_ref[...][:, 0:1] * (s2_ref[0] + s2_ref[1] - hs_ref[...])
           + b_ref[...])
      ystore[pl.ds(i * _BN, _BN), :] = y
      stats[0:1, :] += jnp.sum(y, axis=0, keepdims=True)
      stats[1:2, :] += jnp.sum(y * y, axis=0, keepdims=True)

    @pl.when(i >= nb)
    def _():
      @pl.when(i == nb)
      def _():
        mean = stats[0:1, :] / n
        var = stats[1:2, :] / n - mean * mean
        stats[2:3, :] = mean
        stats[3:4, :] = lax.rsqrt(var + 1e-5)

      j = i - nb
      y = ystore[pl.ds(j * _BN, _BN), :]
      xp = jnp.maximum((y - stats[2:3, :]) * stats[3:4, :] * g_ref[...]
                       + be_ref[...], 0.0)
      oh = (batch_ref[...] == lax.broadcasted_iota(jnp.int32, (1, ng), 1)
            ).astype(jnp.float32)
      psum[...] += lax.dot_general(oh, xp, (((0,), (0,)), ((), ())),
                                   preferred_element_type=jnp.float32)
      ones = jnp.ones((_BN, 1), jnp.float32)
      pcnt[...] += lax.dot_general(oh, ones, (((0,), (0,)), ((), ())),
                                   preferred_element_type=jnp.float32)

      @pl.when(i == 2 * nb - 1)
      def _():
        p = psum[...] / jnp.maximum(pcnt[...], 1.0)
        out_ref[...] = jnp.dot(p, wc_ref[...],
                               preferred_element_type=jnp.float32) + bc_ref[...]

  return pl.pallas_call(
      body,
      grid=(2 * nb,),
      in_specs=[
          pl.BlockSpec((2, _BN, h), lambda i: (0, jnp.minimum(i, nb - 1), 0)),
          pl.BlockSpec((_BN, h), lambda i: (jnp.minimum(i, nb - 1), 0)),
          pl.BlockSpec((_BN, _LW),
                       lambda i: (jnp.where(i < nb, i, i - nb), 0)),
          pl.BlockSpec((1, h), lambda i: (0, 0)),
          pl.BlockSpec((1, h), lambda i: (0, 0)),
          pl.BlockSpec((1, h), lambda i: (0, 0)),
          pl.BlockSpec((_BN, 1), lambda i: (jnp.maximum(i - nb, 0), 0)),
          pl.BlockSpec((h, co), lambda i: (0, 0)),
          pl.BlockSpec((1, co), lambda i: (0, 0)),
      ],
      out_specs=pl.BlockSpec((ng, co), lambda i: (0, 0)),
      out_shape=jax.ShapeDtypeStruct((ng, co), jnp.float32),
      scratch_shapes=[
          pltpu.VMEM((n, h), jnp.float32),
          pltpu.VMEM((8, h), jnp.float32),
          pltpu.VMEM((ng, h), jnp.float32),
          pltpu.VMEM((ng, h), jnp.float32),
      ],
  )(s2, hs, dis, b, g, be, batch2, wc, bc)


def kernel(x, edge_index, batch, W1, b1, g1, be1, W2, b2, g2, be2,
           W3, b3, g3, be3, Wc, bc):
  n, d = x.shape
  h = W1.shape[1]
  src_i = edge_index[0]
  dst_i = edge_index[1]

  ones_tab = jnp.ones((n, h), jnp.float32)
  degp = _sc_count(ones_tab, dst_i).reshape(_NC, n, h)
  hs1, dis = _tc_prep(degp, x, W1)

  s1 = _sc_scatter(hs1, src_i, dst_i).reshape(_NC, n, h)
  hs2 = _tc_layer(s1, hs1, dis, b1.reshape(1, h), g1.reshape(1, h),
                  be1.reshape(1, h), W2)

  s2m = _sc_scatter(hs2, src_i, dst_i).reshape(_NC, n, h)
  hs3 = _tc_layer(s2m, hs2, dis, b2.reshape(1, h), g2.reshape(1, h),
                  be2.reshape(1, h), W3)

  s3m = _sc_scatter(hs3, src_i, dst_i).reshape(_NC, n, h)
  logits = _tc_final(s3m, hs3, dis, b3.reshape(1, h), g3.reshape(1, h),
                     be3.reshape(1, h), batch.reshape(n, 1), Wc,
                     bc.reshape(1, Wc.shape[1]))
  return logits
